# Initial kernel scaffold; baseline (speedup 1.0000x reference)
import jax
import jax.numpy as jnp
from jax.experimental import pallas as pl


def kernel(x, edge_index, edge_weight, W1, b1, W2, b2):
    raise NotImplementedError("write your pallas kernel here")


# speedup vs baseline: 12.5333x; 12.5333x over previous
import functools

import jax
import jax.numpy as jnp
from jax import lax
from jax.experimental import pallas as pl
from jax.experimental.pallas import tpu as pltpu
from jax.experimental.pallas import tpu_sc as plsc

_N = 10000
_NP = 10240
_E = 320000
_F = 128
_L = 64
_NC = 2
_NS = 16
_NW = _NC * _NS
_CH = 128
_CPT = 80
_EP = _NW * _CPT * _CH
_STRIPE = _NP // _NS
_RB = 2000
_GRID = _N // _RB

_f32 = jnp.float32


def _mesh():
    return plsc.VectorSubcoreMesh(core_axis_name="c", subcore_axis_name="s")


_SC_PARAMS = pltpu.CompilerParams(use_tc_tiling_on_sc=False)


def _sc_deg_body(dst_hbm, ew_hbm, out_hbm, dst_v, ew_v, zb, acc):
    c = lax.axis_index("c")
    s = lax.axis_index("s")
    w = s * _NC + c
    pltpu.sync_copy(dst_hbm.at[w], dst_v)
    pltpu.sync_copy(ew_hbm.at[w], ew_v)
    z16 = jnp.zeros((16,), _f32)
    for i in range(_STRIPE // 16):
        zb[pl.ds(i * 16, 16)] = z16
    pltpu.sync_copy(zb, acc.at[pl.ds(s * _STRIPE, _STRIPE)])
    plsc.subcore_barrier()

    def chunk(j, carry):
        pltpu.sync_copy(ew_v.at[j], acc.at[dst_v.at[j]], add=True)
        return carry

    lax.fori_loop(0, _CPT, chunk, 0)
    plsc.subcore_barrier()
    sl = pl.ds(s * _STRIPE, _STRIPE)
    pltpu.sync_copy(acc.at[sl], out_hbm.at[c, sl])


_sc_deg = functools.partial(
    pl.kernel,
    out_type=jax.ShapeDtypeStruct((_NC, _NP), _f32),
    mesh=_mesh(),
    compiler_params=_SC_PARAMS,
    scratch_types=[
        pltpu.VMEM((_CPT, _CH), jnp.int32),
        pltpu.VMEM((_CPT, _CH), _f32),
        pltpu.VMEM((_STRIPE,), _f32),
        pltpu.VMEM_SHARED((_NP,), _f32),
    ],
)(_sc_deg_body)


def _sc_agg_body(q_hbm, src_hbm, dst_hbm, ew_hbm, out_hbm,
                 src_v, dst_v, ew_v, rows, acc, sem):
    c = lax.axis_index("c")
    s = lax.axis_index("s")
    w = s * _NC + c
    pltpu.sync_copy(src_hbm.at[w], src_v)
    pltpu.sync_copy(dst_hbm.at[w], dst_v)
    pltpu.sync_copy(ew_hbm.at[w], ew_v)
    z16 = jnp.zeros((16,), _f32)
    for r in range(_CH):
        for k in range(_L // 16):
            rows[r, pl.ds(k * 16, 16)] = z16
    for i in range(_STRIPE // _CH):
        pltpu.sync_copy(rows, acc.at[pl.ds(s * _STRIPE + i * _CH, _CH)])
    plsc.subcore_barrier()

    def chunk(j, carry):
        pltpu.async_copy(q_hbm.at[src_v.at[j]], rows, sem).wait()
        for g in range(_CH // 16):
            ewv = ew_v[j, pl.ds(g * 16, 16)]
            for t in range(16):
                e = g * 16 + t
                wsc = ewv[t]
                for k in range(_L // 16):
                    sl = pl.ds(k * 16, 16)
                    rows[e, sl] = rows[e, sl] * wsc
        pltpu.sync_copy(rows, acc.at[dst_v.at[j]], add=True)
        return carry

    lax.fori_loop(0, _CPT, chunk, 0)
    plsc.subcore_barrier()
    for i in range(_STRIPE // _CH):
        sl = pl.ds(s * _STRIPE + i * _CH, _CH)
        pltpu.sync_copy(acc.at[sl], out_hbm.at[c, sl])


_sc_agg = functools.partial(
    pl.kernel,
    out_type=jax.ShapeDtypeStruct((_NC, _NP, _L), _f32),
    mesh=_mesh(),
    compiler_params=_SC_PARAMS,
    scratch_types=[
        pltpu.VMEM((_CPT, _CH), jnp.int32),
        pltpu.VMEM((_CPT, _CH), jnp.int32),
        pltpu.VMEM((_CPT, _CH), _f32),
        pltpu.VMEM((_CH, _L), _f32),
        pltpu.VMEM_SHARED((_NP, _L), _f32),
        pltpu.SemaphoreType.DMA,
    ],
)(_sc_agg_body)


def _tc_b_body(degpt_ref, x_ref, w1_ref, dinv_ref, q1_ref):
    deg = degpt_ref[:, 0:1] + degpt_ref[:, 1:2] + 1.0
    dinv = lax.rsqrt(deg)
    dinv_ref[...] = dinv
    h = jnp.dot(x_ref[...], w1_ref[...], preferred_element_type=_f32)
    q1_ref[...] = h * dinv


_tc_b = pl.pallas_call(
    _tc_b_body,
    grid=(_GRID,),
    in_specs=[
        pl.BlockSpec((_RB, 2), lambda i: (i, 0)),
        pl.BlockSpec((_RB, _F), lambda i: (i, 0)),
        pl.BlockSpec((_F, _L), lambda i: (0, 0)),
    ],
    out_specs=[
        pl.BlockSpec((_RB, 1), lambda i: (i, 0)),
        pl.BlockSpec((_RB, _L), lambda i: (i, 0)),
    ],
    out_shape=[
        jax.ShapeDtypeStruct((_N, 1), _f32),
        jax.ShapeDtypeStruct((_N, _L), _f32),
    ],
)


def _tc_d_body(aggp_ref, q1_ref, dinv_ref, b1_ref, q2_ref):
    t = aggp_ref[...]
    ssum = t[0] + t[1] + q1_ref[...]
    dinv = dinv_ref[...]
    z = jnp.maximum(dinv * ssum + b1_ref[...], 0.0)
    q2_ref[...] = dinv * z


_tc_d = pl.pallas_call(
    _tc_d_body,
    grid=(_GRID,),
    in_specs=[
        pl.BlockSpec((_NC, _RB, _L), lambda i: (0, i, 0)),
        pl.BlockSpec((_RB, _L), lambda i: (i, 0)),
        pl.BlockSpec((_RB, 1), lambda i: (i, 0)),
        pl.BlockSpec((1, _L), lambda i: (0, 0)),
    ],
    out_specs=pl.BlockSpec((_RB, _L), lambda i: (i, 0)),
    out_shape=jax.ShapeDtypeStruct((_N, _L), _f32),
)


def _tc_f_body(aggp_ref, q2_ref, dinv_ref, w2_ref, b2_ref, out_ref):
    t = aggp_ref[...]
    agg = dinv_ref[...] * (t[0] + t[1] + q2_ref[...])
    out_ref[...] = jnp.dot(agg, w2_ref[...],
                           preferred_element_type=_f32) + b2_ref[...]


_tc_f = pl.pallas_call(
    _tc_f_body,
    grid=(_GRID,),
    in_specs=[
        pl.BlockSpec((_NC, _RB, _L), lambda i: (0, i, 0)),
        pl.BlockSpec((_RB, _L), lambda i: (i, 0)),
        pl.BlockSpec((_RB, 1), lambda i: (i, 0)),
        pl.BlockSpec((_L, _F), lambda i: (0, 0)),
        pl.BlockSpec((1, _F), lambda i: (0, 0)),
    ],
    out_specs=pl.BlockSpec((_RB, _F), lambda i: (i, 0)),
    out_shape=jax.ShapeDtypeStruct((_N, _F), _f32),
)


@jax.jit
def kernel(x, edge_index, edge_weight, W1, b1, W2, b2):
    src = edge_index[0].astype(jnp.int32)
    dst = edge_index[1].astype(jnp.int32)
    ew = edge_weight.astype(_f32)
    pad = _EP - _E
    zi = jnp.zeros((pad,), jnp.int32)
    src_p = jnp.concatenate([src, zi]).reshape(_NW, _CPT, _CH)
    dst_p = jnp.concatenate([dst, zi]).reshape(_NW, _CPT, _CH)
    ew_p = jnp.concatenate([ew, jnp.zeros((pad,), _f32)]).reshape(_NW, _CPT, _CH)

    degp = _sc_deg(dst_p, ew_p)
    degpt = degp.T
    dinv, q1 = _tc_b(degpt, x, W1)
    aggp1 = _sc_agg(q1, src_p, dst_p, ew_p)
    q2 = _tc_d(aggp1, q1, dinv, b1.reshape(1, _L))
    aggp2 = _sc_agg(q2, src_p, dst_p, ew_p)
    out = _tc_f(aggp2, q2, dinv, W2, b2.reshape(1, _F))
    return out


# speedup vs baseline: 15.5388x; 1.2398x over previous
import functools

import jax
import jax.numpy as jnp
from jax import lax
from jax.experimental import pallas as pl
from jax.experimental.pallas import tpu as pltpu
from jax.experimental.pallas import tpu_sc as plsc

_N = 10000
_NP = 10240
_E = 320000
_F = 128
_L = 64
_NC = 2
_NS = 16
_NW = _NC * _NS
_CH = 128
_CPT = 80
_EP = _NW * _CPT * _CH
_STRIPE = _NP // _NS
_RB = 2000
_GRID = _N // _RB

_f32 = jnp.float32


def _mesh():
    return plsc.VectorSubcoreMesh(core_axis_name="c", subcore_axis_name="s")


_SC_PARAMS = pltpu.CompilerParams(use_tc_tiling_on_sc=False)


def _sc_deg_body(dst_hbm, ew_hbm, out_hbm, dst_v, ew_v, zb, acc):
    c = lax.axis_index("c")
    s = lax.axis_index("s")
    w = s * _NC + c
    pltpu.sync_copy(dst_hbm.at[w], dst_v)
    pltpu.sync_copy(ew_hbm.at[w], ew_v)
    z16 = jnp.zeros((16,), _f32)
    for i in range(_STRIPE // 16):
        zb[pl.ds(i * 16, 16)] = z16
    pltpu.sync_copy(zb, acc.at[pl.ds(s * _STRIPE, _STRIPE)])
    plsc.subcore_barrier()

    def chunk(j, carry):
        pltpu.sync_copy(ew_v.at[j], acc.at[dst_v.at[j]], add=True)
        return carry

    lax.fori_loop(0, _CPT, chunk, 0)
    plsc.subcore_barrier()
    sl = pl.ds(s * _STRIPE, _STRIPE)
    pltpu.sync_copy(acc.at[sl], out_hbm.at[c, sl])


_sc_deg = functools.partial(
    pl.kernel,
    out_type=jax.ShapeDtypeStruct((_NC, _NP), _f32),
    mesh=_mesh(),
    compiler_params=_SC_PARAMS,
    scratch_types=[
        pltpu.VMEM((_CPT, _CH), jnp.int32),
        pltpu.VMEM((_CPT, _CH), _f32),
        pltpu.VMEM((_STRIPE,), _f32),
        pltpu.VMEM_SHARED((_NP,), _f32),
    ],
)(_sc_deg_body)


_NBUF = 6
_GAHEAD = 4
_SLAG = 2


def _sc_agg_body(q_hbm, src_hbm, dst_hbm, ew_hbm, out_hbm,
                 src_v, dst_v, ew_v, bufs, acc, sem_g, sem_s):
    c = lax.axis_index("c")
    s = lax.axis_index("s")
    w = s * _NC + c
    pltpu.sync_copy(src_hbm.at[w], src_v)
    pltpu.sync_copy(dst_hbm.at[w], dst_v)
    pltpu.sync_copy(ew_hbm.at[w], ew_v)
    z16 = jnp.zeros((16,), _f32)
    for r in range(_CH):
        for k in range(_L // 16):
            bufs[0, r, pl.ds(k * 16, 16)] = z16
    for i in range(_STRIPE // _CH):
        pltpu.sync_copy(bufs.at[0], acc.at[pl.ds(s * _STRIPE + i * _CH, _CH)])
    plsc.subcore_barrier()

    for t in range(_GAHEAD):
        pltpu.async_copy(q_hbm.at[src_v.at[t]], bufs.at[t], sem_g)

    def chunk(j, carry):
        m = lax.rem(j, _NBUF)
        pltpu.make_async_copy(q_hbm.at[src_v.at[j]], bufs.at[m], sem_g).wait()
        for g in range(_CH // 16):
            ewv = ew_v[j, pl.ds(g * 16, 16)]
            for t in range(16):
                e = g * 16 + t
                wsc = ewv[t]
                for k in range(_L // 16):
                    sl = pl.ds(k * 16, 16)
                    bufs[m, e, sl] = bufs[m, e, sl] * wsc
        pltpu.async_copy(bufs.at[m], acc.at[dst_v.at[j]], sem_s, add=True)

        @pl.when(j >= _SLAG)
        def _():
            pltpu.make_async_copy(q_hbm.at[src_v.at[j]], bufs.at[m],
                                  sem_s).wait()

        @pl.when(j + _GAHEAD < _CPT)
        def _():
            m4 = lax.rem(j + _GAHEAD, _NBUF)
            pltpu.async_copy(q_hbm.at[src_v.at[j + _GAHEAD]], bufs.at[m4],
                             sem_g)

        return carry

    lax.fori_loop(0, _CPT, chunk, 0)
    for _ in range(_SLAG):
        pltpu.make_async_copy(q_hbm.at[src_v.at[0]], bufs.at[0], sem_s).wait()
    plsc.subcore_barrier()
    for i in range(_STRIPE // _CH):
        sl = pl.ds(s * _STRIPE + i * _CH, _CH)
        pltpu.sync_copy(acc.at[sl], out_hbm.at[c, sl])


_sc_agg = functools.partial(
    pl.kernel,
    out_type=jax.ShapeDtypeStruct((_NC, _NP, _L), _f32),
    mesh=_mesh(),
    compiler_params=_SC_PARAMS,
    scratch_types=[
        pltpu.VMEM((_CPT, _CH), jnp.int32),
        pltpu.VMEM((_CPT, _CH), jnp.int32),
        pltpu.VMEM((_CPT, _CH), _f32),
        pltpu.VMEM((_NBUF, _CH, _L), _f32),
        pltpu.VMEM_SHARED((_NP, _L), _f32),
        pltpu.SemaphoreType.DMA,
        pltpu.SemaphoreType.DMA,
    ],
)(_sc_agg_body)


def _tc_b_body(degpt_ref, x_ref, w1_ref, dinv_ref, q1_ref):
    deg = degpt_ref[:, 0:1] + degpt_ref[:, 1:2] + 1.0
    dinv = lax.rsqrt(deg)
    dinv_ref[...] = dinv
    h = jnp.dot(x_ref[...], w1_ref[...], preferred_element_type=_f32)
    q1_ref[...] = h * dinv


_tc_b = pl.pallas_call(
    _tc_b_body,
    grid=(_GRID,),
    in_specs=[
        pl.BlockSpec((_RB, 2), lambda i: (i, 0)),
        pl.BlockSpec((_RB, _F), lambda i: (i, 0)),
        pl.BlockSpec((_F, _L), lambda i: (0, 0)),
    ],
    out_specs=[
        pl.BlockSpec((_RB, 1), lambda i: (i, 0)),
        pl.BlockSpec((_RB, _L), lambda i: (i, 0)),
    ],
    out_shape=[
        jax.ShapeDtypeStruct((_N, 1), _f32),
        jax.ShapeDtypeStruct((_N, _L), _f32),
    ],
)


def _tc_d_body(aggp_ref, q1_ref, dinv_ref, b1_ref, q2_ref):
    t = aggp_ref[...]
    ssum = t[0] + t[1] + q1_ref[...]
    dinv = dinv_ref[...]
    z = jnp.maximum(dinv * ssum + b1_ref[...], 0.0)
    q2_ref[...] = dinv * z


_tc_d = pl.pallas_call(
    _tc_d_body,
    grid=(_GRID,),
    in_specs=[
        pl.BlockSpec((_NC, _RB, _L), lambda i: (0, i, 0)),
        pl.BlockSpec((_RB, _L), lambda i: (i, 0)),
        pl.BlockSpec((_RB, 1), lambda i: (i, 0)),
        pl.BlockSpec((1, _L), lambda i: (0, 0)),
    ],
    out_specs=pl.BlockSpec((_RB, _L), lambda i: (i, 0)),
    out_shape=jax.ShapeDtypeStruct((_N, _L), _f32),
)


def _tc_f_body(aggp_ref, q2_ref, dinv_ref, w2_ref, b2_ref, out_ref):
    t = aggp_ref[...]
    agg = dinv_ref[...] * (t[0] + t[1] + q2_ref[...])
    out_ref[...] = jnp.dot(agg, w2_ref[...],
                           preferred_element_type=_f32) + b2_ref[...]


_tc_f = pl.pallas_call(
    _tc_f_body,
    grid=(_GRID,),
    in_specs=[
        pl.BlockSpec((_NC, _RB, _L), lambda i: (0, i, 0)),
        pl.BlockSpec((_RB, _L), lambda i: (i, 0)),
        pl.BlockSpec((_RB, 1), lambda i: (i, 0)),
        pl.BlockSpec((_L, _F), lambda i: (0, 0)),
        pl.BlockSpec((1, _F), lambda i: (0, 0)),
    ],
    out_specs=pl.BlockSpec((_RB, _F), lambda i: (i, 0)),
    out_shape=jax.ShapeDtypeStruct((_N, _F), _f32),
)


@jax.jit
def kernel(x, edge_index, edge_weight, W1, b1, W2, b2):
    src = edge_index[0].astype(jnp.int32)
    dst = edge_index[1].astype(jnp.int32)
    ew = edge_weight.astype(_f32)
    pad = _EP - _E
    zi = jnp.zeros((pad,), jnp.int32)
    src_p = jnp.concatenate([src, zi]).reshape(_NW, _CPT, _CH)
    dst_p = jnp.concatenate([dst, zi]).reshape(_NW, _CPT, _CH)
    ew_p = jnp.concatenate([ew, jnp.zeros((pad,), _f32)]).reshape(_NW, _CPT, _CH)

    degp = _sc_deg(dst_p, ew_p)
    degpt = degp.T
    dinv, q1 = _tc_b(degpt, x, W1)
    aggp1 = _sc_agg(q1, src_p, dst_p, ew_p)
    q2 = _tc_d(aggp1, q1, dinv, b1.reshape(1, _L))
    aggp2 = _sc_agg(q2, src_p, dst_p, ew_p)
    out = _tc_f(aggp2, q2, dinv, W2, b2.reshape(1, _F))
    return out


# speedup vs baseline: 24.5732x; 1.5814x over previous
import functools

import jax
import jax.numpy as jnp
from jax import lax
from jax.experimental import pallas as pl
from jax.experimental.pallas import tpu as pltpu
from jax.experimental.pallas import tpu_sc as plsc

_N = 10000
_NP = 10240
_E = 320000
_F = 128
_L = 64
_L2 = _L // 2
_NC = 2
_NS = 16
_NW = _NC * _NS
_CH = 128
_CPT = 80
_EP = _NW * _CPT * _CH
_STRIPE = _NP // _NS
_QSTRIPE = _N // _NS
_RB = 2000
_GRID = _N // _RB
_NBUF = 6
_GAHEAD = 4
_SLAG = 2

_f32 = jnp.float32


def _mesh():
    return plsc.VectorSubcoreMesh(core_axis_name="c", subcore_axis_name="s")


_SC_PARAMS = pltpu.CompilerParams(use_tc_tiling_on_sc=False)


def _sc_deg_body(dst_hbm, ew_hbm, out_hbm, dst_v, ew_v, zb, acc):
    c = lax.axis_index("c")
    s = lax.axis_index("s")
    w = s * _NC + c
    pltpu.sync_copy(dst_hbm.at[w], dst_v)
    pltpu.sync_copy(ew_hbm.at[w], ew_v)
    z16 = jnp.zeros((16,), _f32)
    for i in range(_STRIPE // 16):
        zb[pl.ds(i * 16, 16)] = z16
    pltpu.sync_copy(zb, acc.at[pl.ds(s * _STRIPE, _STRIPE)])
    plsc.subcore_barrier()

    def chunk(j, carry):
        pltpu.sync_copy(ew_v.at[j], acc.at[dst_v.at[j]], add=True)
        return carry

    lax.fori_loop(0, _CPT, chunk, 0)
    plsc.subcore_barrier()
    sl = pl.ds(s * _STRIPE, _STRIPE)
    pltpu.sync_copy(acc.at[sl], out_hbm.at[c, sl])


_sc_deg = functools.partial(
    pl.kernel,
    out_type=jax.ShapeDtypeStruct((_NC, _NP), _f32),
    mesh=_mesh(),
    compiler_params=_SC_PARAMS,
    scratch_types=[
        pltpu.VMEM((_CPT, _CH), jnp.int32),
        pltpu.VMEM((_CPT, _CH), _f32),
        pltpu.VMEM((_STRIPE,), _f32),
        pltpu.VMEM_SHARED((_NP,), _f32),
    ],
)(_sc_deg_body)


def _sc_agg_body(q_hbm, src_hbm, dst_hbm, ew_hbm, out_hbm,
                 src_v, dst_v, ew_v, bufs, qs, acc, sem_g, sem_s):
    c = lax.axis_index("c")
    s = lax.axis_index("s")
    w = s * _NC + c
    pltpu.sync_copy(src_hbm.at[w], src_v)
    pltpu.sync_copy(dst_hbm.at[w], dst_v)
    pltpu.sync_copy(ew_hbm.at[w], ew_v)
    z16 = jnp.zeros((16,), _f32)
    for p in range(2):
        qsl = pl.ds(s * _QSTRIPE, _QSTRIPE)
        pltpu.sync_copy(q_hbm.at[p, qsl], qs.at[qsl])
        for r in range(_CH):
            for k in range(_L2 // 16):
                bufs[0, r, pl.ds(k * 16, 16)] = z16
        for i in range(_STRIPE // _CH):
            pltpu.sync_copy(bufs.at[0],
                            acc.at[pl.ds(s * _STRIPE + i * _CH, _CH)])
        plsc.subcore_barrier()

        for t in range(_GAHEAD):
            pltpu.async_copy(qs.at[src_v.at[t]], bufs.at[t], sem_g)

        def chunk(j, carry):
            m = lax.rem(j, _NBUF)
            pltpu.make_async_copy(qs.at[src_v.at[j]], bufs.at[m],
                                  sem_g).wait()
            for g in range(_CH // 16):
                ewv = ew_v[j, pl.ds(g * 16, 16)]
                for t in range(16):
                    e = g * 16 + t
                    wsc = ewv[t]
                    for k in range(_L2 // 16):
                        sl = pl.ds(k * 16, 16)
                        bufs[m, e, sl] = bufs[m, e, sl] * wsc
            pltpu.async_copy(bufs.at[m], acc.at[dst_v.at[j]], sem_s, add=True)

            @pl.when(j >= _SLAG)
            def _():
                pltpu.make_async_copy(qs.at[src_v.at[j]], bufs.at[m],
                                      sem_s).wait()

            @pl.when(j + _GAHEAD < _CPT)
            def _():
                m4 = lax.rem(j + _GAHEAD, _NBUF)
                pltpu.async_copy(qs.at[src_v.at[j + _GAHEAD]], bufs.at[m4],
                                 sem_g)

            return carry

        lax.fori_loop(0, _CPT, chunk, 0)
        for _ in range(_SLAG):
            pltpu.make_async_copy(qs.at[src_v.at[0]], bufs.at[0],
                                  sem_s).wait()
        plsc.subcore_barrier()
        for i in range(_STRIPE // _CH):
            sl = pl.ds(s * _STRIPE + i * _CH, _CH)
            pltpu.sync_copy(acc.at[sl], out_hbm.at[c, p, sl])


_sc_agg = functools.partial(
    pl.kernel,
    out_type=jax.ShapeDtypeStruct((_NC, 2, _NP, _L2), _f32),
    mesh=_mesh(),
    compiler_params=_SC_PARAMS,
    scratch_types=[
        pltpu.VMEM((_CPT, _CH), jnp.int32),
        pltpu.VMEM((_CPT, _CH), jnp.int32),
        pltpu.VMEM((_CPT, _CH), _f32),
        pltpu.VMEM((_NBUF, _CH, _L2), _f32),
        pltpu.VMEM_SHARED((_N, _L2), _f32),
        pltpu.VMEM_SHARED((_NP, _L2), _f32),
        pltpu.SemaphoreType.DMA,
        pltpu.SemaphoreType.DMA,
    ],
)(_sc_agg_body)


def _split_q(q_ref, val):
    q_ref[0] = val[:, :_L2]
    q_ref[1] = val[:, _L2:]


def _join(t):
    return jnp.concatenate([t[0], t[1]], axis=-1)


def _tc_b_body(degpt_ref, x_ref, w1_ref, dinv_ref, q1_ref):
    deg = degpt_ref[:, 0:1] + degpt_ref[:, 1:2] + 1.0
    dinv = lax.rsqrt(deg)
    dinv_ref[...] = dinv
    h = jnp.dot(x_ref[...], w1_ref[...], preferred_element_type=_f32)
    _split_q(q1_ref, h * dinv)


_tc_b = pl.pallas_call(
    _tc_b_body,
    grid=(_GRID,),
    in_specs=[
        pl.BlockSpec((_RB, 2), lambda i: (i, 0)),
        pl.BlockSpec((_RB, _F), lambda i: (i, 0)),
        pl.BlockSpec((_F, _L), lambda i: (0, 0)),
    ],
    out_specs=[
        pl.BlockSpec((_RB, 1), lambda i: (i, 0)),
        pl.BlockSpec((2, _RB, _L2), lambda i: (0, i, 0)),
    ],
    out_shape=[
        jax.ShapeDtypeStruct((_N, 1), _f32),
        jax.ShapeDtypeStruct((2, _N, _L2), _f32),
    ],
)


def _tc_d_body(aggp_ref, q1_ref, dinv_ref, b1_ref, q2_ref):
    t = aggp_ref[...]
    ssum = _join(t[0] + t[1]) + _join(q1_ref[...])
    dinv = dinv_ref[...]
    z = jnp.maximum(dinv * ssum + b1_ref[...], 0.0)
    _split_q(q2_ref, dinv * z)


_tc_d = pl.pallas_call(
    _tc_d_body,
    grid=(_GRID,),
    in_specs=[
        pl.BlockSpec((_NC, 2, _RB, _L2), lambda i: (0, 0, i, 0)),
        pl.BlockSpec((2, _RB, _L2), lambda i: (0, i, 0)),
        pl.BlockSpec((_RB, 1), lambda i: (i, 0)),
        pl.BlockSpec((1, _L), lambda i: (0, 0)),
    ],
    out_specs=pl.BlockSpec((2, _RB, _L2), lambda i: (0, i, 0)),
    out_shape=jax.ShapeDtypeStruct((2, _N, _L2), _f32),
)


def _tc_f_body(aggp_ref, q2_ref, dinv_ref, w2_ref, b2_ref, out_ref):
    t = aggp_ref[...]
    agg = dinv_ref[...] * (_join(t[0] + t[1]) + _join(q2_ref[...]))
    out_ref[...] = jnp.dot(agg, w2_ref[...],
                           preferred_element_type=_f32) + b2_ref[...]


_tc_f = pl.pallas_call(
    _tc_f_body,
    grid=(_GRID,),
    in_specs=[
        pl.BlockSpec((_NC, 2, _RB, _L2), lambda i: (0, 0, i, 0)),
        pl.BlockSpec((2, _RB, _L2), lambda i: (0, i, 0)),
        pl.BlockSpec((_RB, 1), lambda i: (i, 0)),
        pl.BlockSpec((_L, _F), lambda i: (0, 0)),
        pl.BlockSpec((1, _F), lambda i: (0, 0)),
    ],
    out_specs=pl.BlockSpec((_RB, _F), lambda i: (i, 0)),
    out_shape=jax.ShapeDtypeStruct((_N, _F), _f32),
)


@jax.jit
def kernel(x, edge_index, edge_weight, W1, b1, W2, b2):
    src = edge_index[0].astype(jnp.int32)
    dst = edge_index[1].astype(jnp.int32)
    ew = edge_weight.astype(_f32)
    pad = _EP - _E
    zi = jnp.zeros((pad,), jnp.int32)
    src_p = jnp.concatenate([src, zi]).reshape(_NW, _CPT, _CH)
    dst_p = jnp.concatenate([dst, zi]).reshape(_NW, _CPT, _CH)
    ew_p = jnp.concatenate([ew, jnp.zeros((pad,), _f32)]).reshape(_NW, _CPT, _CH)

    degp = _sc_deg(dst_p, ew_p)
    degpt = degp.T
    dinv, q1 = _tc_b(degpt, x, W1)
    aggp1 = _sc_agg(q1, src_p, dst_p, ew_p)
    q2 = _tc_d(aggp1, q1, dinv, b1.reshape(1, _L))
    aggp2 = _sc_agg(q2, src_p, dst_p, ew_p)
    out = _tc_f(aggp2, q2, dinv, W2, b2.reshape(1, _F))
    return out


# speedup vs baseline: 30.8384x; 1.2550x over previous
import functools

import jax
import jax.numpy as jnp
from jax import lax
from jax.experimental import pallas as pl
from jax.experimental.pallas import tpu as pltpu
from jax.experimental.pallas import tpu_sc as plsc

_N = 10000
_NP = 10240
_E = 320000
_F = 128
_L = 64
_L2 = _L // 2
_NC = 2
_NS = 16
_NW = _NC * _NS
_CH = 256
_CPT = 40
_EP = _NW * _CPT * _CH
_STRIPE = _NP // _NS
_QSTRIPE = _N // _NS
_RB = 2000
_GRID = _N // _RB
_NBUF = 6
_GAHEAD = 4
_SLAG = 2

_f32 = jnp.float32


def _mesh():
    return plsc.VectorSubcoreMesh(core_axis_name="c", subcore_axis_name="s")


_SC_PARAMS = pltpu.CompilerParams(use_tc_tiling_on_sc=False)


def _sc_deg_body(dst_hbm, ew_hbm, out_hbm, dst_v, ew_v, zb, acc):
    c = lax.axis_index("c")
    s = lax.axis_index("s")
    w = s * _NC + c
    pltpu.sync_copy(dst_hbm.at[w], dst_v)
    pltpu.sync_copy(ew_hbm.at[w], ew_v)
    z16 = jnp.zeros((16,), _f32)
    for i in range(_STRIPE // 16):
        zb[pl.ds(i * 16, 16)] = z16
    pltpu.sync_copy(zb, acc.at[pl.ds(s * _STRIPE, _STRIPE)])
    plsc.subcore_barrier()

    def chunk(j, carry):
        pltpu.sync_copy(ew_v.at[j], acc.at[dst_v.at[j]], add=True)
        return carry

    lax.fori_loop(0, _CPT, chunk, 0)
    plsc.subcore_barrier()
    sl = pl.ds(s * _STRIPE, _STRIPE)
    pltpu.sync_copy(acc.at[sl], out_hbm.at[c, sl])


_sc_deg = functools.partial(
    pl.kernel,
    out_type=jax.ShapeDtypeStruct((_NC, _NP), _f32),
    mesh=_mesh(),
    compiler_params=_SC_PARAMS,
    scratch_types=[
        pltpu.VMEM((_CPT, _CH), jnp.int32),
        pltpu.VMEM((_CPT, _CH), _f32),
        pltpu.VMEM((_STRIPE,), _f32),
        pltpu.VMEM_SHARED((_NP,), _f32),
    ],
)(_sc_deg_body)


def _sc_agg_body(q_hbm, src_hbm, dst_hbm, ew_hbm, out_hbm,
                 src_v, dst_v, ew_v, bufs, qs, acc, sem_g, sem_s):
    c = lax.axis_index("c")
    s = lax.axis_index("s")
    w = s * _NC + c
    pltpu.sync_copy(src_hbm.at[w], src_v)
    pltpu.sync_copy(dst_hbm.at[w], dst_v)
    pltpu.sync_copy(ew_hbm.at[w], ew_v)
    z16 = jnp.zeros((16,), _f32)
    for p in range(2):
        qsl = pl.ds(s * _QSTRIPE, _QSTRIPE)
        pltpu.sync_copy(q_hbm.at[p, qsl], qs.at[qsl])
        for r in range(_CH):
            for k in range(_L2 // 16):
                bufs[0, r, pl.ds(k * 16, 16)] = z16
        for i in range(_STRIPE // _CH):
            pltpu.sync_copy(bufs.at[0],
                            acc.at[pl.ds(s * _STRIPE + i * _CH, _CH)])
        _REM = _STRIPE - (_STRIPE // _CH) * _CH
        if _REM:
            pltpu.sync_copy(
                bufs.at[0, pl.ds(0, _REM)],
                acc.at[pl.ds(s * _STRIPE + (_STRIPE // _CH) * _CH, _REM)])
        plsc.subcore_barrier()

        for t in range(_GAHEAD):
            pltpu.async_copy(qs.at[src_v.at[t]], bufs.at[t], sem_g)

        def chunk(j, carry):
            m = lax.rem(j, _NBUF)
            pltpu.make_async_copy(qs.at[src_v.at[j]], bufs.at[m],
                                  sem_g).wait()
            for g in range(_CH // 16):
                ewv = ew_v[j, pl.ds(g * 16, 16)]
                for t in range(16):
                    e = g * 16 + t
                    wsc = ewv[t]
                    for k in range(_L2 // 16):
                        sl = pl.ds(k * 16, 16)
                        bufs[m, e, sl] = bufs[m, e, sl] * wsc
            pltpu.async_copy(bufs.at[m], acc.at[dst_v.at[j]], sem_s, add=True)

            @pl.when(j >= _SLAG)
            def _():
                pltpu.make_async_copy(qs.at[src_v.at[j]], bufs.at[m],
                                      sem_s).wait()

            @pl.when(j + _GAHEAD < _CPT)
            def _():
                m4 = lax.rem(j + _GAHEAD, _NBUF)
                pltpu.async_copy(qs.at[src_v.at[j + _GAHEAD]], bufs.at[m4],
                                 sem_g)

            return carry

        lax.fori_loop(0, _CPT, chunk, 0)
        for _ in range(_SLAG):
            pltpu.make_async_copy(qs.at[src_v.at[0]], bufs.at[0],
                                  sem_s).wait()
        plsc.subcore_barrier()
        for i in range(_STRIPE // _CH):
            sl = pl.ds(s * _STRIPE + i * _CH, _CH)
            pltpu.sync_copy(acc.at[sl], out_hbm.at[c, p, sl])
        if _REM:
            sl = pl.ds(s * _STRIPE + (_STRIPE // _CH) * _CH, _REM)
            pltpu.sync_copy(acc.at[sl], out_hbm.at[c, p, sl])


_sc_agg = functools.partial(
    pl.kernel,
    out_type=jax.ShapeDtypeStruct((_NC, 2, _NP, _L2), _f32),
    mesh=_mesh(),
    compiler_params=_SC_PARAMS,
    scratch_types=[
        pltpu.VMEM((_CPT, _CH), jnp.int32),
        pltpu.VMEM((_CPT, _CH), jnp.int32),
        pltpu.VMEM((_CPT, _CH), _f32),
        pltpu.VMEM((_NBUF, _CH, _L2), _f32),
        pltpu.VMEM_SHARED((_N, _L2), _f32),
        pltpu.VMEM_SHARED((_NP, _L2), _f32),
        pltpu.SemaphoreType.DMA,
        pltpu.SemaphoreType.DMA,
    ],
)(_sc_agg_body)


def _split_q(q_ref, val):
    q_ref[0] = val[:, :_L2]
    q_ref[1] = val[:, _L2:]


def _join(t):
    return jnp.concatenate([t[0], t[1]], axis=-1)


def _tc_b_body(degpt_ref, x_ref, w1_ref, dinv_ref, q1_ref):
    deg = degpt_ref[:, 0:1] + degpt_ref[:, 1:2] + 1.0
    dinv = lax.rsqrt(deg)
    dinv_ref[...] = dinv
    h = jnp.dot(x_ref[...], w1_ref[...], preferred_element_type=_f32)
    _split_q(q1_ref, h * dinv)


_tc_b = pl.pallas_call(
    _tc_b_body,
    grid=(_GRID,),
    in_specs=[
        pl.BlockSpec((_RB, 2), lambda i: (i, 0)),
        pl.BlockSpec((_RB, _F), lambda i: (i, 0)),
        pl.BlockSpec((_F, _L), lambda i: (0, 0)),
    ],
    out_specs=[
        pl.BlockSpec((_RB, 1), lambda i: (i, 0)),
        pl.BlockSpec((2, _RB, _L2), lambda i: (0, i, 0)),
    ],
    out_shape=[
        jax.ShapeDtypeStruct((_N, 1), _f32),
        jax.ShapeDtypeStruct((2, _N, _L2), _f32),
    ],
)


def _tc_d_body(aggp_ref, q1_ref, dinv_ref, b1_ref, q2_ref):
    t = aggp_ref[...]
    ssum = _join(t[0] + t[1]) + _join(q1_ref[...])
    dinv = dinv_ref[...]
    z = jnp.maximum(dinv * ssum + b1_ref[...], 0.0)
    _split_q(q2_ref, dinv * z)


_tc_d = pl.pallas_call(
    _tc_d_body,
    grid=(_GRID,),
    in_specs=[
        pl.BlockSpec((_NC, 2, _RB, _L2), lambda i: (0, 0, i, 0)),
        pl.BlockSpec((2, _RB, _L2), lambda i: (0, i, 0)),
        pl.BlockSpec((_RB, 1), lambda i: (i, 0)),
        pl.BlockSpec((1, _L), lambda i: (0, 0)),
    ],
    out_specs=pl.BlockSpec((2, _RB, _L2), lambda i: (0, i, 0)),
    out_shape=jax.ShapeDtypeStruct((2, _N, _L2), _f32),
)


def _tc_f_body(aggp_ref, q2_ref, dinv_ref, w2_ref, b2_ref, out_ref):
    t = aggp_ref[...]
    agg = dinv_ref[...] * (_join(t[0] + t[1]) + _join(q2_ref[...]))
    out_ref[...] = jnp.dot(agg, w2_ref[...],
                           preferred_element_type=_f32) + b2_ref[...]


_tc_f = pl.pallas_call(
    _tc_f_body,
    grid=(_GRID,),
    in_specs=[
        pl.BlockSpec((_NC, 2, _RB, _L2), lambda i: (0, 0, i, 0)),
        pl.BlockSpec((2, _RB, _L2), lambda i: (0, i, 0)),
        pl.BlockSpec((_RB, 1), lambda i: (i, 0)),
        pl.BlockSpec((_L, _F), lambda i: (0, 0)),
        pl.BlockSpec((1, _F), lambda i: (0, 0)),
    ],
    out_specs=pl.BlockSpec((_RB, _F), lambda i: (i, 0)),
    out_shape=jax.ShapeDtypeStruct((_N, _F), _f32),
)


@jax.jit
def kernel(x, edge_index, edge_weight, W1, b1, W2, b2):
    src = edge_index[0].astype(jnp.int32)
    dst = edge_index[1].astype(jnp.int32)
    ew = edge_weight.astype(_f32)
    pad = _EP - _E
    zi = jnp.zeros((pad,), jnp.int32)
    src_p = jnp.concatenate([src, zi]).reshape(_NW, _CPT, _CH)
    dst_p = jnp.concatenate([dst, zi]).reshape(_NW, _CPT, _CH)
    ew_p = jnp.concatenate([ew, jnp.zeros((pad,), _f32)]).reshape(_NW, _CPT, _CH)

    degp = _sc_deg(dst_p, ew_p)
    degpt = degp.T
    dinv, q1 = _tc_b(degpt, x, W1)
    aggp1 = _sc_agg(q1, src_p, dst_p, ew_p)
    q2 = _tc_d(aggp1, q1, dinv, b1.reshape(1, _L))
    aggp2 = _sc_agg(q2, src_p, dst_p, ew_p)
    out = _tc_f(aggp2, q2, dinv, W2, b2.reshape(1, _F))
    return out
